# baseline (speedup 1.0000x reference)
import functools

import jax
import jax.numpy as jnp
from jax import lax
from jax.experimental import pallas as pl
from jax.experimental.pallas import tpu as pltpu
from jax.experimental.pallas import tpu_sc as plsc

BATCH = 16384
NC = 2
NS = 16
L = 16
NW = NC * NS
B_PER_W = BATCH // NW
TABLE_ROWS = 1000000
TABLE_PAD = 448


def _gather_body(idx_hbm, table_hbm, out_hbm, idx_v, rows_v, sem):
    wid = lax.axis_index("s") * NC + lax.axis_index("c")
    base = wid * B_PER_W
    pltpu.sync_copy(idx_hbm.at[pl.ds(base, B_PER_W)], idx_v)
    pltpu.async_copy(table_hbm.at[idx_v], rows_v, sem).wait()
    pltpu.sync_copy(rows_v, out_hbm.at[pl.ds(base, B_PER_W)])


@jax.jit
def kernel(inputs, categ_bias, output_layer_bias):
    idx = inputs[:, 0].astype(jnp.int32)
    biased = categ_bias + output_layer_bias[0, 0]
    table = jnp.pad(biased, ((0, TABLE_PAD), (0, 0))).reshape(-1)

    mesh = plsc.VectorSubcoreMesh(core_axis_name="c", subcore_axis_name="s")
    run = pl.kernel(
        _gather_body,
        mesh=mesh,
        out_type=jax.ShapeDtypeStruct((BATCH,), jnp.float32),
        scratch_types=[
            pltpu.VMEM((B_PER_W,), jnp.int32),
            pltpu.VMEM((B_PER_W,), jnp.float32),
            pltpu.SemaphoreType.DMA,
        ],
    )
    out = run(idx, table)
    return out.reshape(BATCH, 1)


# speedup vs baseline: 1.1244x; 1.1244x over previous
import functools

import jax
import jax.numpy as jnp
from jax import lax
from jax.experimental import pallas as pl
from jax.experimental.pallas import tpu as pltpu
from jax.experimental.pallas import tpu_sc as plsc

BATCH = 16384
NC = 2
NS = 16
L = 16
NW = NC * NS
B_PER_W = BATCH // NW
TABLE_ROWS = 1000000
TABLE_PAD = 448


def _gather_body(idx_hbm, table_hbm, out_hbm, idx_v, rows_v, sem):
    wid = lax.axis_index("s") * NC + lax.axis_index("c")
    base = wid * B_PER_W
    pltpu.sync_copy(idx_hbm.at[pl.ds(base, B_PER_W)], idx_v)
    pltpu.async_copy(table_hbm.at[idx_v], rows_v, sem).wait()
    pltpu.sync_copy(rows_v, out_hbm.at[pl.ds(base, B_PER_W)])


@jax.jit
def kernel(inputs, categ_bias, output_layer_bias):
    idx = inputs[:, 0].astype(jnp.int32)
    padded = jnp.concatenate(
        [categ_bias, jnp.zeros((TABLE_PAD, 1), jnp.float32)], axis=0
    )
    table = (padded + output_layer_bias[0, 0]).reshape(-1)

    mesh = plsc.VectorSubcoreMesh(core_axis_name="c", subcore_axis_name="s")
    run = pl.kernel(
        _gather_body,
        mesh=mesh,
        out_type=jax.ShapeDtypeStruct((BATCH,), jnp.float32),
        scratch_types=[
            pltpu.VMEM((B_PER_W,), jnp.int32),
            pltpu.VMEM((B_PER_W,), jnp.float32),
            pltpu.SemaphoreType.DMA,
        ],
    )
    out = run(idx, table)
    return out.reshape(BATCH, 1)


# speedup vs baseline: 1.1260x; 1.0014x over previous
import functools

import jax
import jax.numpy as jnp
from jax import lax
from jax.experimental import pallas as pl
from jax.experimental.pallas import tpu as pltpu
from jax.experimental.pallas import tpu_sc as plsc

BATCH = 16384
NC = 2
NS = 16
L = 16
NW = NC * NS
B_PER_W = BATCH // NW
TABLE_ROWS = 1000000
TABLE_PAD = 448


HALF = B_PER_W // 2


def _gather_body(idx_hbm, table_hbm, out_hbm, idx_a, idx_b, rows_a, rows_b,
                 sem_ia, sem_ib, sem_a, sem_b):
    wid = lax.axis_index("s") * NC + lax.axis_index("c")
    base = wid * B_PER_W
    ia = pltpu.async_copy(idx_hbm.at[pl.ds(base, HALF)], idx_a, sem_ia)
    ib = pltpu.async_copy(idx_hbm.at[pl.ds(base + HALF, HALF)], idx_b, sem_ib)
    ia.wait()
    ga = pltpu.async_copy(table_hbm.at[idx_a], rows_a, sem_a)
    ib.wait()
    gb = pltpu.async_copy(table_hbm.at[idx_b], rows_b, sem_b)
    ga.wait()
    ca = pltpu.async_copy(rows_a, out_hbm.at[pl.ds(base, HALF)], sem_ia)
    gb.wait()
    cb = pltpu.async_copy(rows_b, out_hbm.at[pl.ds(base + HALF, HALF)], sem_ib)
    ca.wait()
    cb.wait()


@jax.jit
def kernel(inputs, categ_bias, output_layer_bias):
    idx = inputs[:, 0].astype(jnp.int32)
    padded = jnp.concatenate(
        [categ_bias, jnp.zeros((TABLE_PAD, 1), jnp.float32)], axis=0
    )
    table = (padded + output_layer_bias[0, 0]).reshape(-1)

    mesh = plsc.VectorSubcoreMesh(core_axis_name="c", subcore_axis_name="s")
    run = pl.kernel(
        _gather_body,
        mesh=mesh,
        out_type=jax.ShapeDtypeStruct((BATCH,), jnp.float32),
        scratch_types=[
            pltpu.VMEM((HALF,), jnp.int32),
            pltpu.VMEM((HALF,), jnp.int32),
            pltpu.VMEM((HALF,), jnp.float32),
            pltpu.VMEM((HALF,), jnp.float32),
            pltpu.SemaphoreType.DMA,
            pltpu.SemaphoreType.DMA,
            pltpu.SemaphoreType.DMA,
            pltpu.SemaphoreType.DMA,
        ],
    )
    out = run(idx, table)
    return out.reshape(BATCH, 1)
